# Initial kernel scaffold; baseline (speedup 1.0000x reference)
import jax
import jax.numpy as jnp
from jax.experimental import pallas as pl


def kernel(feat, edge_index, W, b):
    raise NotImplementedError("write your pallas kernel here")


# speedup vs baseline: 19.5426x; 19.5426x over previous
import functools

import jax
import jax.numpy as jnp
from jax import lax
from jax.experimental import pallas as pl
from jax.experimental.pallas import tpu as pltpu
from jax.experimental.pallas import tpu_sc as plsc

_N = 10000
_E = 320000
_D = 128
_C = 64
_K = 8
_ALPHA = 0.05

_NSUB = 16
_NCORE = 2
_CH = 128
_NCH = 160
_EPT = _NCH * _CH
_EP = _NSUB * _EPT
_RPT = 640
_NB = _RPT // _CH
_NR = _NSUB * _RPT
_CHALF = _C // _NCORE

_CKS = [(1.0 - _ALPHA) * (1.0 / _K) ** (_K + 1 - k) for k in range(1, _K + 1)]
_BETA = _ALPHA * sum((1.0 / _K) ** j for j in range(1, _K + 1))


def _project_body(f_ref, w_ref, o_ref):
    o_ref[...] = lax.dot_general(
        f_ref[...], w_ref[...],
        dimension_numbers=(((1,), (1,)), ((), ())),
        preferred_element_type=jnp.float32,
        precision=lax.Precision.HIGHEST,
    )


def _propagate_body(y0_hbm, src_hbm, dst_hbm, b_hbm, out_hbm,
                    yA, yB, si, di, gb0, gb1, gb2, gb3,
                    acc, zbuf, bv,
                    sg0, sg1, sg2, sg3, ss0, ss1, ss2, ss3):
    c = lax.axis_index("c")
    s = lax.axis_index("s")
    row0 = s * _RPT
    gbufs = (gb0, gb1, gb2, gb3)
    gsems = (sg0, sg1, sg2, sg3)
    ssems = (ss0, ss1, ss2, ss3)

    pltpu.sync_copy(src_hbm.at[s], si)
    pltpu.sync_copy(dst_hbm.at[s], di)
    pltpu.sync_copy(b_hbm.at[pl.ds(c * _CHALF, _CHALF)], bv)

    zv = jnp.zeros((16,), jnp.float32)

    def _zero_row(i, carry):
        zbuf[i, pl.ds(0, 16)] = zv
        zbuf[i, pl.ds(16, 16)] = zv
        return carry

    lax.fori_loop(0, _CH, _zero_row, 0)

    pltpu.sync_copy(y0_hbm.at[c, pl.ds(row0, _RPT)], acc)
    pltpu.sync_copy(acc, yA.at[pl.ds(row0, _RPT)])

    def _zero_blk_init(j, carry):
        pltpu.sync_copy(zbuf, yB.at[pl.ds(row0 + j * _CH, _CH)])
        return carry

    lax.fori_loop(0, _NB, _zero_blk_init, 0)
    plsc.subcore_barrier()

    for k in range(1, _K + 1):
        src_tab, dst_tab = (yA, yB) if k % 2 == 1 else (yB, yA)

        def _edges(t, carry, src_tab=src_tab, dst_tab=dst_tab):
            base = t * 4
            gds = []
            for j in range(4):
                gds.append(pltpu.async_copy(
                    src_tab.at[si.at[base + j]], gbufs[j], gsems[j]))
            sds = []
            for j in range(4):
                gds[j].wait()
                sds.append(pltpu.async_copy(
                    gbufs[j], dst_tab.at[di.at[base + j]], ssems[j],
                    add=True))
            for sd in sds:
                sd.wait()
            return carry

        lax.fori_loop(0, _NCH // 4, _edges, 0)
        plsc.subcore_barrier()

        ck = _CKS[k - 1]
        if k == _K:
            blo = bv[pl.ds(0, 16)]
            bhi = bv[pl.ds(16, 16)]

        def _upd_blk(j, carry, src_tab=src_tab, dst_tab=dst_tab, k=k, ck=ck):
            blk0 = j * _CH
            pltpu.sync_copy(dst_tab.at[pl.ds(row0 + blk0, _CH)], gb0)

            if k == 1:
                def _fma(i, c2):
                    r = blk0 + i
                    for h in (0, 16):
                        acc[r, pl.ds(h, 16)] = (acc[r, pl.ds(h, 16)] * _BETA
                                                + gb0[i, pl.ds(h, 16)] * ck)
                    return c2
            elif k < _K:
                def _fma(i, c2):
                    r = blk0 + i
                    for h in (0, 16):
                        acc[r, pl.ds(h, 16)] = (acc[r, pl.ds(h, 16)]
                                                + gb0[i, pl.ds(h, 16)] * ck)
                    return c2
            else:
                def _fma(i, c2):
                    r = blk0 + i
                    acc[r, pl.ds(0, 16)] = (acc[r, pl.ds(0, 16)]
                                            + gb0[i, pl.ds(0, 16)] * ck + blo)
                    acc[r, pl.ds(16, 16)] = (acc[r, pl.ds(16, 16)]
                                             + gb0[i, pl.ds(16, 16)] * ck
                                             + bhi)
                    return c2

            lax.fori_loop(0, _CH, _fma, carry)
            if k < _K:
                pltpu.sync_copy(zbuf, src_tab.at[pl.ds(row0 + blk0, _CH)])
            return carry

        lax.fori_loop(0, _NB, _upd_blk, 0)
        if k < _K:
            plsc.subcore_barrier()

    pltpu.sync_copy(acc, out_hbm.at[c, s])


_propagate = functools.partial(
    pl.kernel,
    out_type=jax.ShapeDtypeStruct((_NCORE, _NSUB, _RPT, _CHALF), jnp.float32),
    mesh=plsc.VectorSubcoreMesh(
        core_axis_name="c", subcore_axis_name="s",
        num_cores=_NCORE, num_subcores=_NSUB),
    compiler_params=pltpu.CompilerParams(use_tc_tiling_on_sc=False),
    scratch_types=[
        pltpu.VMEM_SHARED((_NR, _CHALF), jnp.float32),
        pltpu.VMEM_SHARED((_NR, _CHALF), jnp.float32),
        pltpu.VMEM((_NCH, _CH), jnp.int32),
        pltpu.VMEM((_NCH, _CH), jnp.int32),
        pltpu.VMEM((_CH, _CHALF), jnp.float32),
        pltpu.VMEM((_CH, _CHALF), jnp.float32),
        pltpu.VMEM((_CH, _CHALF), jnp.float32),
        pltpu.VMEM((_CH, _CHALF), jnp.float32),
        pltpu.VMEM((_RPT, _CHALF), jnp.float32),
        pltpu.VMEM((_CH, _CHALF), jnp.float32),
        pltpu.VMEM((_CHALF,), jnp.float32),
        pltpu.SemaphoreType.DMA, pltpu.SemaphoreType.DMA,
        pltpu.SemaphoreType.DMA, pltpu.SemaphoreType.DMA,
        pltpu.SemaphoreType.DMA, pltpu.SemaphoreType.DMA,
        pltpu.SemaphoreType.DMA, pltpu.SemaphoreType.DMA,
    ],
)(_propagate_body)


def kernel(feat, edge_index, W, b):
    feat_p = jnp.pad(feat, ((0, _NR - _N), (0, 0)))
    y0 = pl.pallas_call(
        _project_body,
        out_shape=jax.ShapeDtypeStruct((_NR, _C), jnp.float32),
    )(feat_p, W)
    y0s = y0.reshape(_NR, _NCORE, _CHALF).transpose(1, 0, 2)

    src = edge_index[0]
    dst = edge_index[1]
    pad_idx = (_N + (jnp.arange(_EP - _E, dtype=jnp.int32) % (_NR - _N)))
    srcs = jnp.concatenate([src, pad_idx]).reshape(_NSUB, _NCH, _CH)
    dsts = jnp.concatenate([dst, pad_idx]).reshape(_NSUB, _NCH, _CH)

    out_sc = _propagate(y0s, srcs, dsts, b)
    return out_sc.transpose(1, 2, 0, 3).reshape(_NR, _C)[:_N]
